# Initial kernel scaffold; baseline (speedup 1.0000x reference)
import jax
import jax.numpy as jnp
from jax.experimental import pallas as pl


def kernel(x, edge_index, W1, a_src1, a_dst1, b1, W2, a_src2, a_dst2, b2):
    raise NotImplementedError("write your pallas kernel here")


# speedup vs baseline: 26.3885x; 26.3885x over previous
import functools

import jax
import jax.numpy as jnp
from jax import lax
from jax.experimental import pallas as pl
from jax.experimental.pallas import tpu as pltpu
from jax.experimental.pallas import tpu_sc as plsc

N = 10000
E = 320000
IN = 128
HID = 32
HEADS = 8
C = 16

NC = 2
NS = 16
L = 16

BN = 400
CH = 80
ROWS_PER_TILE = N // NS
EDGES_PER_TILE_L1 = E // NS
EDGES_PER_TILE_L2 = E // (NC * NS)
HHALF = HEADS // 2
FHALF = HHALF * HID
AUGW = FHALF + L


def _tc1_body(x_ref, w_ref, as_ref, ad_ref, hlo_ref, hhi_ref, dpad_ref):
    h = jnp.dot(x_ref[...], w_ref[...], preferred_element_type=jnp.float32)
    s = jnp.dot(h, as_ref[...], preferred_element_type=jnp.float32)
    d = jnp.dot(h, ad_ref[...], preferred_element_type=jnp.float32)
    z8 = jnp.zeros((BN, HEADS), jnp.float32)
    hlo_ref[...] = jnp.concatenate([h[:, :FHALF], s, z8], axis=1)
    hhi_ref[...] = jnp.concatenate([h[:, FHALF:], s, z8], axis=1)
    dpad_ref[...] = jnp.concatenate([d, z8], axis=1)


def _tc1(x, W1, A1s, A1d):
    grid = (N // BN,)
    return pl.pallas_call(
        _tc1_body,
        grid=grid,
        in_specs=[
            pl.BlockSpec((BN, IN), lambda i: (i, 0)),
            pl.BlockSpec((IN, HEADS * HID), lambda i: (0, 0)),
            pl.BlockSpec((HEADS * HID, HEADS), lambda i: (0, 0)),
            pl.BlockSpec((HEADS * HID, HEADS), lambda i: (0, 0)),
        ],
        out_specs=[
            pl.BlockSpec((BN, AUGW), lambda i: (i, 0)),
            pl.BlockSpec((BN, AUGW), lambda i: (i, 0)),
            pl.BlockSpec((BN, L), lambda i: (i, 0)),
        ],
        out_shape=[
            jax.ShapeDtypeStruct((N, AUGW), jnp.float32),
            jax.ShapeDtypeStruct((N, AUGW), jnp.float32),
            jax.ShapeDtypeStruct((N, L), jnp.float32),
        ],
    )(x, W1, A1s, A1d)


@functools.lru_cache(maxsize=None)
def _mesh():
    return plsc.VectorSubcoreMesh(core_axis_name="c", subcore_axis_name="s",
                                  num_cores=NC, num_subcores=NS)


def _zero_f32(ref, nrows, ncols):
    zero = jnp.zeros((L,), jnp.float32)

    def body(r, _):
        for cc in range(ncols // L):
            ref[r, pl.ds(cc * L, L)] = zero
        return 0

    lax.fori_loop(0, nrows, body, 0)


@functools.lru_cache(maxsize=None)
def _sc1_call():
    return pl.kernel(
        _sc1_body,
        mesh=_mesh(),
        out_type=[
            jax.ShapeDtypeStruct((N, AUGW), jnp.float32),
            jax.ShapeDtypeStruct((N, AUGW), jnp.float32),
            jax.ShapeDtypeStruct((N, L), jnp.float32),
            jax.ShapeDtypeStruct((N, L), jnp.float32),
        ],
        scratch_types=[
            pltpu.VMEM((CH, AUGW), jnp.float32),
            pltpu.VMEM((CH, L), jnp.float32),
            pltpu.VMEM((CH, L), jnp.float32),
            pltpu.VMEM((CH,), jnp.int32),
            pltpu.VMEM((CH,), jnp.int32),
            pltpu.VMEM((25, AUGW), jnp.float32),
            pltpu.VMEM((25, L), jnp.float32),
            pltpu.VMEM_SHARED((N, AUGW), jnp.float32),
            pltpu.VMEM_SHARED((N, L), jnp.float32),
            pltpu.SemaphoreType.DMA,
        ],
        compiler_params=pltpu.CompilerParams(use_tc_tiling_on_sc=False,
                                             needs_layout_passes=False),
    )


def _sc1_body(src_hbm, dst_hbm, hlo_hbm, hhi_hbm, dpad_hbm,
              unlo_hbm, unhi_hbm, denlo_hbm, denhi_hbm,
              rows_v, dt_v, pb_v, srcb, dstb, zb, zb16, acc_h, acc_p, sem):
    cid = lax.axis_index("c")
    sid = lax.axis_index("s")

    _zero_f32(zb, 25, AUGW)
    _zero_f32(zb16, 25, L)
    _zero_f32(pb_v, CH, L)
    rbase = sid * ROWS_PER_TILE
    for k in range(ROWS_PER_TILE // 25):
        pltpu.sync_copy(zb, acc_h.at[pl.ds(rbase + k * 25, 25)])
        pltpu.sync_copy(zb16, acc_p.at[pl.ds(rbase + k * 25, 25)])
    plsc.subcore_barrier()

    ebase = sid * EDGES_PER_TILE_L1
    iota = lax.iota(jnp.int32, L)
    hcol0 = FHALF + cid * HHALF
    dcol0 = cid * HHALF

    def chunk(ci, _):
        off = ebase + ci * CH
        pltpu.sync_copy(src_hbm.at[pl.ds(off, CH)], srcb)
        pltpu.sync_copy(dst_hbm.at[pl.ds(off, CH)], dstb)

        @pl.when(cid == 0)
        def _():
            pltpu.async_copy(hlo_hbm.at[srcb], rows_v, sem).wait()

        @pl.when(cid == 1)
        def _():
            pltpu.async_copy(hhi_hbm.at[srcb], rows_v, sem).wait()

        pltpu.async_copy(dpad_hbm.at[dstb], dt_v, sem).wait()

        def pgrp(g, _):
            row_idx = g * L + iota
            for t in range(HHALF):
                sv = plsc.load_gather(
                    rows_v, [row_idx, jnp.zeros((L,), jnp.int32) + (hcol0 + t)])
                dv = plsc.load_gather(
                    dt_v, [row_idx, jnp.zeros((L,), jnp.int32) + (dcol0 + t)])
                e = sv + dv
                e = jnp.maximum(e, 0.2 * e)
                p = jnp.exp(e)
                plsc.store_scatter(
                    pb_v, [row_idx, jnp.full((L,), t, jnp.int32)], p)
            return 0

        lax.fori_loop(0, CH // L, pgrp, 0)

        def erow(e, _):
            ei = jnp.full((L,), e, jnp.int32)
            for t in range(HHALF):
                pv = plsc.load_gather(pb_v, [ei, jnp.full((L,), t, jnp.int32)])
                for j in range(HID // L):
                    col = t * HID + j * L
                    rows_v[e, pl.ds(col, L)] = rows_v[e, pl.ds(col, L)] * pv
            return 0

        lax.fori_loop(0, CH, erow, 0)

        pltpu.sync_copy(rows_v, acc_h.at[dstb], add=True)
        pltpu.sync_copy(pb_v, acc_p.at[dstb], add=True)
        return 0

    lax.fori_loop(0, EDGES_PER_TILE_L1 // CH, chunk, 0)
    plsc.subcore_barrier()

    @pl.when(cid == 0)
    def _():
        pltpu.sync_copy(acc_h.at[pl.ds(rbase, ROWS_PER_TILE)],
                        unlo_hbm.at[pl.ds(rbase, ROWS_PER_TILE)])
        pltpu.sync_copy(acc_p.at[pl.ds(rbase, ROWS_PER_TILE)],
                        denlo_hbm.at[pl.ds(rbase, ROWS_PER_TILE)])

    @pl.when(cid == 1)
    def _():
        pltpu.sync_copy(acc_h.at[pl.ds(rbase, ROWS_PER_TILE)],
                        unhi_hbm.at[pl.ds(rbase, ROWS_PER_TILE)])
        pltpu.sync_copy(acc_p.at[pl.ds(rbase, ROWS_PER_TILE)],
                        denhi_hbm.at[pl.ds(rbase, ROWS_PER_TILE)])


def _tc2_body(unlo_ref, unhi_ref, denlo_ref, denhi_ref, b1_ref, w2_ref,
              a2_ref, h2_ref):
    segs = []
    for un, den in ((unlo_ref, denlo_ref), (unhi_ref, denhi_ref)):
        u = un[...]
        d = den[...]
        for t in range(HHALF):
            segs.append(u[:, t * HID:(t + 1) * HID]
                        / (d[:, t:t + 1] + 1e-16))
    agg = jnp.concatenate(segs, axis=1) + b1_ref[...]
    out1 = jnp.where(agg > 0, agg, jnp.exp(jnp.minimum(agg, 0.0)) - 1.0)
    h2 = jnp.dot(out1, w2_ref[...], preferred_element_type=jnp.float32)
    a2 = a2_ref[...]
    s2 = jnp.sum(h2 * a2[0:1, :], axis=1, keepdims=True)
    d2 = jnp.sum(h2 * a2[1:2, :], axis=1, keepdims=True)
    z14 = jnp.zeros((BN, C - 2), jnp.float32)
    h2_ref[...] = jnp.concatenate([h2, s2, d2, z14], axis=1)


def _tc2(unlo, unhi, denlo, denhi, b1, W2, a2):
    grid = (N // BN,)
    return pl.pallas_call(
        _tc2_body,
        grid=grid,
        in_specs=[
            pl.BlockSpec((BN, AUGW), lambda i: (i, 0)),
            pl.BlockSpec((BN, AUGW), lambda i: (i, 0)),
            pl.BlockSpec((BN, L), lambda i: (i, 0)),
            pl.BlockSpec((BN, L), lambda i: (i, 0)),
            pl.BlockSpec((1, HEADS * HID), lambda i: (0, 0)),
            pl.BlockSpec((HEADS * HID, C), lambda i: (0, 0)),
            pl.BlockSpec((2, C), lambda i: (0, 0)),
        ],
        out_specs=pl.BlockSpec((BN, 2 * C), lambda i: (i, 0)),
        out_shape=jax.ShapeDtypeStruct((N, 2 * C), jnp.float32),
    )(unlo, unhi, denlo, denhi, b1, W2, a2)


ACC2W = 2 * L


@functools.lru_cache(maxsize=None)
def _sc2_call():
    return pl.kernel(
        _sc2_body,
        mesh=_mesh(),
        out_type=[
            jax.ShapeDtypeStruct((N, ACC2W), jnp.float32),
            jax.ShapeDtypeStruct((N, ACC2W), jnp.float32),
        ],
        scratch_types=[
            pltpu.VMEM((CH, ACC2W), jnp.float32),
            pltpu.VMEM((CH, ACC2W), jnp.float32),
            pltpu.VMEM((CH, ACC2W), jnp.float32),
            pltpu.VMEM((CH,), jnp.int32),
            pltpu.VMEM((CH,), jnp.int32),
            pltpu.VMEM((25, ACC2W), jnp.float32),
            pltpu.VMEM_SHARED((N, ACC2W), jnp.float32),
            pltpu.SemaphoreType.DMA,
        ],
        compiler_params=pltpu.CompilerParams(use_tc_tiling_on_sc=False,
                                             needs_layout_passes=False),
    )


def _sc2_body(src_hbm, dst_hbm, h2_hbm,
              acc_a_hbm, acc_b_hbm,
              rows16_v, sd_v, rows_v, srcb, dstb, zb, acc, sem):
    cid = lax.axis_index("c")
    sid = lax.axis_index("s")

    _zero_f32(zb, 25, ACC2W)
    _zero_f32(rows_v, CH, ACC2W)
    rbase = sid * ROWS_PER_TILE
    for k in range(ROWS_PER_TILE // 25):
        pltpu.sync_copy(zb, acc.at[pl.ds(rbase + k * 25, 25)])
    plsc.subcore_barrier()

    wid = cid * NS + sid
    ebase = wid * EDGES_PER_TILE_L2
    iota = lax.iota(jnp.int32, L)

    def chunk(ci, _):
        off = ebase + ci * CH
        pltpu.sync_copy(src_hbm.at[pl.ds(off, CH)], srcb)
        pltpu.sync_copy(dst_hbm.at[pl.ds(off, CH)], dstb)
        pltpu.async_copy(h2_hbm.at[srcb], rows16_v, sem).wait()
        pltpu.async_copy(h2_hbm.at[dstb], sd_v, sem).wait()

        def pgrp(g, _):
            row_idx = g * L + iota
            sv = plsc.load_gather(rows16_v, [row_idx,
                                             jnp.full((L,), C, jnp.int32)])
            dv = plsc.load_gather(sd_v, [row_idx,
                                         jnp.full((L,), C + 1, jnp.int32)])
            e = sv + dv
            e = jnp.maximum(e, 0.2 * e)
            p = jnp.exp(e)
            plsc.store_scatter(
                rows_v, [row_idx, jnp.full((L,), C, jnp.int32)], p)
            return 0

        lax.fori_loop(0, CH // L, pgrp, 0)

        def erow(e, _):
            ei = jnp.full((L,), e, jnp.int32)
            pv = plsc.load_gather(rows_v, [ei, jnp.full((L,), C, jnp.int32)])
            rows_v[e, pl.ds(0, L)] = rows16_v[e, pl.ds(0, L)] * pv
            return 0

        lax.fori_loop(0, CH, erow, 0)

        pltpu.sync_copy(rows_v, acc.at[dstb], add=True)
        return 0

    lax.fori_loop(0, EDGES_PER_TILE_L2 // CH, chunk, 0)
    plsc.subcore_barrier()

    @pl.when(cid == 0)
    def _():
        pltpu.sync_copy(acc.at[pl.ds(rbase, ROWS_PER_TILE)],
                        acc_a_hbm.at[pl.ds(rbase, ROWS_PER_TILE)])

    @pl.when(cid == 1)
    def _():
        pltpu.sync_copy(acc.at[pl.ds(rbase, ROWS_PER_TILE)],
                        acc_b_hbm.at[pl.ds(rbase, ROWS_PER_TILE)])


def _tc3_body(a_ref, b_ref, b2_ref, out_ref):
    ua = a_ref[...]
    ub = b_ref[...]
    un = ua[:, :C] + ub[:, :C]
    den = ua[:, C:C + 1] + ub[:, C:C + 1]
    o = un / (den + 1e-16) + b2_ref[...]
    m = jnp.max(o, axis=1, keepdims=True)
    lse = m + jnp.log(jnp.sum(jnp.exp(o - m), axis=1, keepdims=True))
    out_ref[...] = o - lse


def _tc3(acc_a, acc_b, b2):
    grid = (N // BN,)
    return pl.pallas_call(
        _tc3_body,
        grid=grid,
        in_specs=[
            pl.BlockSpec((BN, ACC2W), lambda i: (i, 0)),
            pl.BlockSpec((BN, ACC2W), lambda i: (i, 0)),
            pl.BlockSpec((1, C), lambda i: (0, 0)),
        ],
        out_specs=pl.BlockSpec((BN, C), lambda i: (i, 0)),
        out_shape=jax.ShapeDtypeStruct((N, C), jnp.float32),
    )(acc_a, acc_b, b2)


@jax.jit
def kernel(x, edge_index, W1, a_src1, a_dst1, b1, W2, a_src2, a_dst2, b2):
    src = edge_index[0]
    dst = edge_index[1]

    eye = jnp.eye(HEADS, dtype=jnp.float32)
    A1s = (a_src1[:, :, None] * eye[:, None, :]).reshape(HEADS * HID, HEADS)
    A1d = (a_dst1[:, :, None] * eye[:, None, :]).reshape(HEADS * HID, HEADS)

    hlo, hhi, dpad = _tc1(x, W1, A1s, A1d)
    unlo, unhi, denlo, denhi = _sc1_call()(src, dst, hlo, hhi, dpad)
    h2aug = _tc2(unlo, unhi, denlo, denhi, b1.reshape(1, HEADS * HID), W2,
                 jnp.concatenate([a_src2, a_dst2], axis=0))
    acc_a, acc_b = _sc2_call()(src, dst, h2aug)
    return _tc3(acc_a, acc_b, b2.reshape(1, C))


# speedup vs baseline: 39.7936x; 1.5080x over previous
import functools

import jax
import jax.numpy as jnp
from jax import lax
from jax.experimental import pallas as pl
from jax.experimental.pallas import tpu as pltpu
from jax.experimental.pallas import tpu_sc as plsc

N = 10000
E = 320000
IN = 128
HID = 32
HEADS = 8
C = 16

NC = 2
NS = 16
L = 16

BN = 400
CH = 80
ROWS_PER_TILE = N // NS
EDGES_PER_TILE_L1 = E // NS
EDGES_PER_TILE_L2 = E // (NC * NS)
HHALF = HEADS // 2
FHALF = HHALF * HID


def _tc1_body(x_ref, w_ref, as_ref, ad_ref, hlo_ref, hhi_ref, s_ref, d_ref):
    h = jnp.dot(x_ref[...], w_ref[...], preferred_element_type=jnp.float32)
    s = jnp.dot(h, as_ref[...], preferred_element_type=jnp.float32)
    d = jnp.dot(h, ad_ref[...], preferred_element_type=jnp.float32)
    z8 = jnp.zeros((BN, HEADS), jnp.float32)
    hlo_ref[...] = h[:, :FHALF]
    hhi_ref[...] = h[:, FHALF:]
    s_ref[...] = jnp.concatenate([s, z8], axis=1)
    d_ref[...] = jnp.concatenate([d, z8], axis=1)


def _tc1(x, W1, A1s, A1d):
    grid = (N // BN,)
    return pl.pallas_call(
        _tc1_body,
        grid=grid,
        in_specs=[
            pl.BlockSpec((BN, IN), lambda i: (i, 0)),
            pl.BlockSpec((IN, HEADS * HID), lambda i: (0, 0)),
            pl.BlockSpec((HEADS * HID, HEADS), lambda i: (0, 0)),
            pl.BlockSpec((HEADS * HID, HEADS), lambda i: (0, 0)),
        ],
        out_specs=[
            pl.BlockSpec((BN, FHALF), lambda i: (i, 0)),
            pl.BlockSpec((BN, FHALF), lambda i: (i, 0)),
            pl.BlockSpec((BN, L), lambda i: (i, 0)),
            pl.BlockSpec((BN, L), lambda i: (i, 0)),
        ],
        out_shape=[
            jax.ShapeDtypeStruct((N, FHALF), jnp.float32),
            jax.ShapeDtypeStruct((N, FHALF), jnp.float32),
            jax.ShapeDtypeStruct((N, L), jnp.float32),
            jax.ShapeDtypeStruct((N, L), jnp.float32),
        ],
    )(x, W1, A1s, A1d)


@functools.lru_cache(maxsize=None)
def _mesh():
    return plsc.VectorSubcoreMesh(core_axis_name="c", subcore_axis_name="s",
                                  num_cores=NC, num_subcores=NS)


def _zero_f32(ref, nrows, ncols):
    zero = jnp.zeros((L,), jnp.float32)

    def body(r, _):
        for cc in range(ncols // L):
            ref[r, pl.ds(cc * L, L)] = zero
        return 0

    lax.fori_loop(0, nrows, body, 0)


@functools.lru_cache(maxsize=None)
def _sc1_call():
    return pl.kernel(
        _sc1_body,
        mesh=_mesh(),
        out_type=[
            jax.ShapeDtypeStruct((N, FHALF), jnp.float32),
            jax.ShapeDtypeStruct((N, FHALF), jnp.float32),
            jax.ShapeDtypeStruct((N, L), jnp.float32),
            jax.ShapeDtypeStruct((N, L), jnp.float32),
        ],
        scratch_types=[
            pltpu.VMEM((2, CH, FHALF), jnp.float32),
            pltpu.VMEM((2, CH, L), jnp.float32),
            pltpu.VMEM((2, CH, L), jnp.float32),
            pltpu.VMEM((2, CH, L), jnp.float32),
            pltpu.VMEM((2, CH), jnp.int32),
            pltpu.VMEM((2, CH), jnp.int32),
            pltpu.VMEM((25, FHALF), jnp.float32),
            pltpu.VMEM((25, L), jnp.float32),
            pltpu.VMEM_SHARED((N, FHALF), jnp.float32),
            pltpu.VMEM_SHARED((N, L), jnp.float32),
            pltpu.SemaphoreType.DMA,
            pltpu.SemaphoreType.DMA,
        ],
        compiler_params=pltpu.CompilerParams(use_tc_tiling_on_sc=False,
                                             needs_layout_passes=False),
    )


def _sc1_body(src_hbm, dst_hbm, hlo_hbm, hhi_hbm, spad_hbm, dpad_hbm,
              unlo_hbm, unhi_hbm, denlo_hbm, denhi_hbm,
              rows_v, st_v, dt_v, pb_v, srcb, dstb, zb, zb16,
              acc_h, acc_p, sem0, sem1):
    cid = lax.axis_index("c")
    sid = lax.axis_index("s")
    sems = (sem0, sem1)

    _zero_f32(zb, 25, FHALF)
    _zero_f32(zb16, 25, L)
    _zero_f32(pb_v.at[0], CH, L)
    _zero_f32(pb_v.at[1], CH, L)
    rbase = sid * ROWS_PER_TILE
    for k in range(ROWS_PER_TILE // 25):
        pltpu.sync_copy(zb, acc_h.at[pl.ds(rbase + k * 25, 25)])
        pltpu.sync_copy(zb16, acc_p.at[pl.ds(rbase + k * 25, 25)])
    plsc.subcore_barrier()

    ebase = sid * EDGES_PER_TILE_L1
    nchunks = EDGES_PER_TILE_L1 // CH
    iota = lax.iota(jnp.int32, L)
    col0 = cid * HHALF

    def gather_descs(b):
        h_hbm = (hlo_hbm, hhi_hbm)
        descs = [
            pltpu.make_async_copy(spad_hbm.at[srcb.at[b]], st_v.at[b], sems[b]),
            pltpu.make_async_copy(dpad_hbm.at[dstb.at[b]], dt_v.at[b], sems[b]),
        ]
        return descs, h_hbm

    def load_chunk(b, ci):
        off = ebase + ci * CH
        pltpu.sync_copy(src_hbm.at[pl.ds(off, CH)], srcb.at[b])
        pltpu.sync_copy(dst_hbm.at[pl.ds(off, CH)], dstb.at[b])
        descs, h_hbm = gather_descs(b)
        for d in descs:
            d.start()

        @pl.when(cid == 0)
        def _():
            pltpu.make_async_copy(h_hbm[0].at[srcb.at[b]], rows_v.at[b],
                                  sems[b]).start()

        @pl.when(cid == 1)
        def _():
            pltpu.make_async_copy(h_hbm[1].at[srcb.at[b]], rows_v.at[b],
                                  sems[b]).start()

    def wait_chunk(b):
        descs, h_hbm = gather_descs(b)
        for d in descs:
            d.wait()
        pltpu.make_async_copy(h_hbm[0].at[srcb.at[b]], rows_v.at[b],
                              sems[b]).wait()

    def compute_chunk(b):
        rv = rows_v.at[b]
        pv_ref = pb_v.at[b]
        sv_ref = st_v.at[b]
        dv_ref = dt_v.at[b]

        def pgrp(g, _):
            row_idx = g * L + iota
            for t in range(HHALF):
                colv = jnp.zeros((L,), jnp.int32) + (col0 + t)
                sv = plsc.load_gather(sv_ref, [row_idx, colv])
                dv = plsc.load_gather(dv_ref, [row_idx, colv])
                e = sv + dv
                e = jnp.maximum(e, 0.2 * e)
                p = jnp.exp(e)
                plsc.store_scatter(
                    pv_ref, [row_idx, jnp.full((L,), t, jnp.int32)], p)
            return 0

        lax.fori_loop(0, CH // L, pgrp, 0)

        def erow(e, _):
            ei = jnp.full((L,), e, jnp.int32)
            for t in range(HHALF):
                pv = plsc.load_gather(pv_ref, [ei, jnp.full((L,), t, jnp.int32)])
                for j in range(HID // L):
                    col = t * HID + j * L
                    rv[e, pl.ds(col, L)] = rv[e, pl.ds(col, L)] * pv
            return 0

        lax.fori_loop(0, CH, erow, 0)

    def scatter_chunk(b):
        pltpu.sync_copy(rows_v.at[b], acc_h.at[dstb.at[b]], add=True)
        pltpu.sync_copy(pb_v.at[b], acc_p.at[dstb.at[b]], add=True)

    load_chunk(0, 0)

    def pair(k, _):
        for b in range(2):
            ci = 2 * k + b
            wait_chunk(b)

            @pl.when(ci + 1 < nchunks)
            def _():
                load_chunk(1 - b, ci + 1)

            compute_chunk(b)
            scatter_chunk(b)
        return 0

    lax.fori_loop(0, nchunks // 2, pair, 0)
    plsc.subcore_barrier()

    @pl.when(cid == 0)
    def _():
        pltpu.sync_copy(acc_h.at[pl.ds(rbase, ROWS_PER_TILE)],
                        unlo_hbm.at[pl.ds(rbase, ROWS_PER_TILE)])
        pltpu.sync_copy(acc_p.at[pl.ds(rbase, ROWS_PER_TILE)],
                        denlo_hbm.at[pl.ds(rbase, ROWS_PER_TILE)])

    @pl.when(cid == 1)
    def _():
        pltpu.sync_copy(acc_h.at[pl.ds(rbase, ROWS_PER_TILE)],
                        unhi_hbm.at[pl.ds(rbase, ROWS_PER_TILE)])
        pltpu.sync_copy(acc_p.at[pl.ds(rbase, ROWS_PER_TILE)],
                        denhi_hbm.at[pl.ds(rbase, ROWS_PER_TILE)])


def _tc2_body(unlo_ref, unhi_ref, denlo_ref, denhi_ref, b1_ref, w2_ref,
              a2_ref, h2_ref):
    segs = []
    for un, den in ((unlo_ref, denlo_ref), (unhi_ref, denhi_ref)):
        u = un[...]
        d = den[...]
        for t in range(HHALF):
            segs.append(u[:, t * HID:(t + 1) * HID]
                        / (d[:, t:t + 1] + 1e-16))
    agg = jnp.concatenate(segs, axis=1) + b1_ref[...]
    out1 = jnp.where(agg > 0, agg, jnp.exp(jnp.minimum(agg, 0.0)) - 1.0)
    h2 = jnp.dot(out1, w2_ref[...], preferred_element_type=jnp.float32)
    a2 = a2_ref[...]
    s2 = jnp.sum(h2 * a2[0:1, :], axis=1, keepdims=True)
    d2 = jnp.sum(h2 * a2[1:2, :], axis=1, keepdims=True)
    z14 = jnp.zeros((BN, C - 2), jnp.float32)
    h2_ref[...] = jnp.concatenate([h2, s2, d2, z14], axis=1)


def _tc2(unlo, unhi, denlo, denhi, b1, W2, a2):
    grid = (N // BN,)
    return pl.pallas_call(
        _tc2_body,
        grid=grid,
        in_specs=[
            pl.BlockSpec((BN, FHALF), lambda i: (i, 0)),
            pl.BlockSpec((BN, FHALF), lambda i: (i, 0)),
            pl.BlockSpec((BN, L), lambda i: (i, 0)),
            pl.BlockSpec((BN, L), lambda i: (i, 0)),
            pl.BlockSpec((1, HEADS * HID), lambda i: (0, 0)),
            pl.BlockSpec((HEADS * HID, C), lambda i: (0, 0)),
            pl.BlockSpec((2, C), lambda i: (0, 0)),
        ],
        out_specs=pl.BlockSpec((BN, 2 * C), lambda i: (i, 0)),
        out_shape=jax.ShapeDtypeStruct((N, 2 * C), jnp.float32),
    )(unlo, unhi, denlo, denhi, b1, W2, a2)


ACC2W = 2 * L


@functools.lru_cache(maxsize=None)
def _sc2_call():
    return pl.kernel(
        _sc2_body,
        mesh=_mesh(),
        out_type=[
            jax.ShapeDtypeStruct((N, ACC2W), jnp.float32),
            jax.ShapeDtypeStruct((N, ACC2W), jnp.float32),
        ],
        scratch_types=[
            pltpu.VMEM((2, CH, ACC2W), jnp.float32),
            pltpu.VMEM((2, CH, ACC2W), jnp.float32),
            pltpu.VMEM((2, CH, ACC2W), jnp.float32),
            pltpu.VMEM((2, CH), jnp.int32),
            pltpu.VMEM((2, CH), jnp.int32),
            pltpu.VMEM((25, ACC2W), jnp.float32),
            pltpu.VMEM_SHARED((N, ACC2W), jnp.float32),
            pltpu.SemaphoreType.DMA,
            pltpu.SemaphoreType.DMA,
        ],
        compiler_params=pltpu.CompilerParams(use_tc_tiling_on_sc=False,
                                             needs_layout_passes=False),
    )


def _sc2_body(src_hbm, dst_hbm, h2_hbm,
              acc_a_hbm, acc_b_hbm,
              srows_v, drows_v, rows_v, srcb, dstb, zb, acc, sem0, sem1):
    cid = lax.axis_index("c")
    sid = lax.axis_index("s")
    sems = (sem0, sem1)

    _zero_f32(zb, 25, ACC2W)
    _zero_f32(rows_v.at[0], CH, ACC2W)
    _zero_f32(rows_v.at[1], CH, ACC2W)
    rbase = sid * ROWS_PER_TILE
    for k in range(ROWS_PER_TILE // 25):
        pltpu.sync_copy(zb, acc.at[pl.ds(rbase + k * 25, 25)])
    plsc.subcore_barrier()

    wid = cid * NS + sid
    ebase = wid * EDGES_PER_TILE_L2
    nchunks = EDGES_PER_TILE_L2 // CH
    iota = lax.iota(jnp.int32, L)

    def descs(b):
        return (
            pltpu.make_async_copy(h2_hbm.at[srcb.at[b]], srows_v.at[b], sems[b]),
            pltpu.make_async_copy(h2_hbm.at[dstb.at[b]], drows_v.at[b], sems[b]),
        )

    def load_chunk(b, ci):
        off = ebase + ci * CH
        pltpu.sync_copy(src_hbm.at[pl.ds(off, CH)], srcb.at[b])
        pltpu.sync_copy(dst_hbm.at[pl.ds(off, CH)], dstb.at[b])
        for d in descs(b):
            d.start()

    def wait_chunk(b):
        for d in descs(b):
            d.wait()

    def compute_chunk(b):
        sr = srows_v.at[b]
        dr = drows_v.at[b]
        rv = rows_v.at[b]

        def pgrp(g, _):
            row_idx = g * L + iota
            sv = plsc.load_gather(sr, [row_idx, jnp.full((L,), C, jnp.int32)])
            dv = plsc.load_gather(dr, [row_idx, jnp.full((L,), C + 1, jnp.int32)])
            e = sv + dv
            e = jnp.maximum(e, 0.2 * e)
            p = jnp.exp(e)
            plsc.store_scatter(rv, [row_idx, jnp.full((L,), C, jnp.int32)], p)
            return 0

        lax.fori_loop(0, CH // L, pgrp, 0)

        def erow(e, _):
            ei = jnp.full((L,), e, jnp.int32)
            pv = plsc.load_gather(rv, [ei, jnp.full((L,), C, jnp.int32)])
            rv[e, pl.ds(0, L)] = sr[e, pl.ds(0, L)] * pv
            return 0

        lax.fori_loop(0, CH, erow, 0)

    def scatter_chunk(b):
        pltpu.sync_copy(rows_v.at[b], acc.at[dstb.at[b]], add=True)

    load_chunk(0, 0)

    def pair(k, _):
        for b in range(2):
            ci = 2 * k + b
            wait_chunk(b)

            @pl.when(ci + 1 < nchunks)
            def _():
                load_chunk(1 - b, ci + 1)

            compute_chunk(b)
            scatter_chunk(b)
        return 0

    lax.fori_loop(0, nchunks // 2, pair, 0)
    if nchunks % 2:
        wait_chunk(0)
        compute_chunk(0)
        scatter_chunk(0)
    plsc.subcore_barrier()

    @pl.when(cid == 0)
    def _():
        pltpu.sync_copy(acc.at[pl.ds(rbase, ROWS_PER_TILE)],
                        acc_a_hbm.at[pl.ds(rbase, ROWS_PER_TILE)])

    @pl.when(cid == 1)
    def _():
        pltpu.sync_copy(acc.at[pl.ds(rbase, ROWS_PER_TILE)],
                        acc_b_hbm.at[pl.ds(rbase, ROWS_PER_TILE)])


def _tc3_body(a_ref, b_ref, b2_ref, out_ref):
    ua = a_ref[...]
    ub = b_ref[...]
    un = ua[:, :C] + ub[:, :C]
    den = ua[:, C:C + 1] + ub[:, C:C + 1]
    o = un / (den + 1e-16) + b2_ref[...]
    m = jnp.max(o, axis=1, keepdims=True)
    lse = m + jnp.log(jnp.sum(jnp.exp(o - m), axis=1, keepdims=True))
    out_ref[...] = o - lse


def _tc3(acc_a, acc_b, b2):
    grid = (N // BN,)
    return pl.pallas_call(
        _tc3_body,
        grid=grid,
        in_specs=[
            pl.BlockSpec((BN, ACC2W), lambda i: (i, 0)),
            pl.BlockSpec((BN, ACC2W), lambda i: (i, 0)),
            pl.BlockSpec((1, C), lambda i: (0, 0)),
        ],
        out_specs=pl.BlockSpec((BN, C), lambda i: (i, 0)),
        out_shape=jax.ShapeDtypeStruct((N, C), jnp.float32),
    )(acc_a, acc_b, b2)


@jax.jit
def kernel(x, edge_index, W1, a_src1, a_dst1, b1, W2, a_src2, a_dst2, b2):
    src = edge_index[0]
    dst = edge_index[1]

    eye = jnp.eye(HEADS, dtype=jnp.float32)
    A1s = (a_src1[:, :, None] * eye[:, None, :]).reshape(HEADS * HID, HEADS)
    A1d = (a_dst1[:, :, None] * eye[:, None, :]).reshape(HEADS * HID, HEADS)

    hlo, hhi, spad, dpad = _tc1(x, W1, A1s, A1d)
    unlo, unhi, denlo, denhi = _sc1_call()(src, dst, hlo, hhi, spad, dpad)
    h2aug = _tc2(unlo, unhi, denlo, denhi, b1.reshape(1, HEADS * HID), W2,
                 jnp.concatenate([a_src2, a_dst2], axis=0))
    acc_a, acc_b = _sc2_call()(src, dst, h2aug)
    return _tc3(acc_a, acc_b, b2.reshape(1, C))


# speedup vs baseline: 52.8083x; 1.3271x over previous
import functools

import jax
import jax.numpy as jnp
from jax import lax
from jax.experimental import pallas as pl
from jax.experimental.pallas import tpu as pltpu
from jax.experimental.pallas import tpu_sc as plsc

N = 10000
E = 320000
IN = 128
HID = 32
HEADS = 8
C = 16

NC = 2
NS = 16
L = 16

BN = 400
CH = 80
SUP = 25
ROWS_PER_TILE = N // NS
EDGES_PER_TILE_L1 = E // NS
EDGES_PER_TILE_L2 = E // (NC * NS)
HHALF = HEADS // 2
FHALF = HHALF * HID


def _tc1_body(x_ref, w_ref, as_ref, ad_ref, hlo_ref, hhi_ref, s_ref, d_ref):
    h = jnp.dot(x_ref[...], w_ref[...], preferred_element_type=jnp.float32)
    s = jnp.dot(h, as_ref[...], preferred_element_type=jnp.float32)
    d = jnp.dot(h, ad_ref[...], preferred_element_type=jnp.float32)
    z8 = jnp.zeros((BN, HEADS), jnp.float32)
    hlo_ref[...] = h[:, :FHALF]
    hhi_ref[...] = h[:, FHALF:]
    s_ref[...] = jnp.concatenate([s, z8], axis=1)
    d_ref[...] = jnp.concatenate([d, z8], axis=1)


def _tc1(x, W1, A1s, A1d):
    grid = (N // BN,)
    return pl.pallas_call(
        _tc1_body,
        grid=grid,
        in_specs=[
            pl.BlockSpec((BN, IN), lambda i: (i, 0)),
            pl.BlockSpec((IN, HEADS * HID), lambda i: (0, 0)),
            pl.BlockSpec((HEADS * HID, HEADS), lambda i: (0, 0)),
            pl.BlockSpec((HEADS * HID, HEADS), lambda i: (0, 0)),
        ],
        out_specs=[
            pl.BlockSpec((BN, FHALF), lambda i: (i, 0)),
            pl.BlockSpec((BN, FHALF), lambda i: (i, 0)),
            pl.BlockSpec((BN, L), lambda i: (i, 0)),
            pl.BlockSpec((BN, L), lambda i: (i, 0)),
        ],
        out_shape=[
            jax.ShapeDtypeStruct((N, FHALF), jnp.float32),
            jax.ShapeDtypeStruct((N, FHALF), jnp.float32),
            jax.ShapeDtypeStruct((N, L), jnp.float32),
            jax.ShapeDtypeStruct((N, L), jnp.float32),
        ],
    )(x, W1, A1s, A1d)


@functools.lru_cache(maxsize=None)
def _mesh():
    return plsc.VectorSubcoreMesh(core_axis_name="c", subcore_axis_name="s",
                                  num_cores=NC, num_subcores=NS)


def _zero_f32(ref, nrows, ncols):
    zero = jnp.zeros((L,), jnp.float32)

    def body(r, _):
        for cc in range(ncols // L):
            ref[r, pl.ds(cc * L, L)] = zero
        return 0

    lax.fori_loop(0, nrows, body, 0)


@functools.lru_cache(maxsize=None)
def _sc1_call():
    return pl.kernel(
        _sc1_body,
        mesh=_mesh(),
        out_type=[
            jax.ShapeDtypeStruct((N, FHALF), jnp.float32),
            jax.ShapeDtypeStruct((N, FHALF), jnp.float32),
            jax.ShapeDtypeStruct((N, L), jnp.float32),
            jax.ShapeDtypeStruct((N, L), jnp.float32),
        ],
        scratch_types=[
            pltpu.VMEM((2, CH, FHALF), jnp.float32),
            pltpu.VMEM((2, CH, L), jnp.float32),
            pltpu.VMEM((2, CH, L), jnp.float32),
            pltpu.VMEM((2, CH, L), jnp.float32),
            pltpu.VMEM((2, SUP, CH), jnp.int32),
            pltpu.VMEM((2, SUP, CH), jnp.int32),
            pltpu.VMEM_SHARED((N, FHALF), jnp.float32),
            pltpu.VMEM_SHARED((N, L), jnp.float32),
            pltpu.SemaphoreType.DMA,
            pltpu.SemaphoreType.DMA,
            pltpu.SemaphoreType.DMA,
            pltpu.SemaphoreType.DMA,
        ],
        compiler_params=pltpu.CompilerParams(use_tc_tiling_on_sc=False,
                                             needs_layout_passes=False),
    )


def _sc1_body(src_hbm, dst_hbm, hlo_hbm, hhi_hbm, spad_hbm, dpad_hbm,
              unlo_hbm, unhi_hbm, denlo_hbm, denhi_hbm,
              rows_v, st_v, dt_v, pb_v, srcb, dstb,
              acc_h, acc_p, sem0, sem1, sem2, sem3):
    cid = lax.axis_index("c")
    sid = lax.axis_index("s")
    sems = (sem0, sem1)
    ssems = (sem2, sem3)

    _zero_f32(pb_v.at[0], CH, L)
    _zero_f32(pb_v.at[1], CH, L)
    _zero_f32(rows_v.at[0], CH, FHALF)
    rbase = sid * ROWS_PER_TILE
    for k in range(ROWS_PER_TILE // 25):
        pltpu.sync_copy(rows_v.at[0, pl.ds(0, 25)],
                        acc_h.at[pl.ds(rbase + k * 25, 25)])
        pltpu.sync_copy(pb_v.at[0, pl.ds(0, 25)],
                        acc_p.at[pl.ds(rbase + k * 25, 25)])
    plsc.subcore_barrier()

    crow0 = sid * (EDGES_PER_TILE_L1 // CH)
    nchunks = EDGES_PER_TILE_L1 // CH
    iota = lax.iota(jnp.int32, L)
    col0 = cid * HHALF

    def load_super(m):
        row = crow0 + m * SUP
        s = lax.rem(m, 2)
        pltpu.sync_copy(src_hbm.at[pl.ds(row, SUP)], srcb.at[s])
        pltpu.sync_copy(dst_hbm.at[pl.ds(row, SUP)], dstb.at[s])

    def load_chunk(b, ci):
        j = lax.rem(ci, SUP)
        s = lax.rem(lax.div(ci, SUP), 2)
        si = srcb.at[s, j]
        di = dstb.at[s, j]
        pltpu.make_async_copy(spad_hbm.at[si], st_v.at[b], sems[b]).start()
        pltpu.make_async_copy(dpad_hbm.at[di], dt_v.at[b], sems[b]).start()

        @pl.when(cid == 0)
        def _():
            pltpu.make_async_copy(hlo_hbm.at[si], rows_v.at[b],
                                  sems[b]).start()

        @pl.when(cid == 1)
        def _():
            pltpu.make_async_copy(hhi_hbm.at[si], rows_v.at[b],
                                  sems[b]).start()

    def wait_chunk(b):
        pltpu.make_async_copy(spad_hbm.at[srcb.at[0, 0]], st_v.at[b],
                              sems[b]).wait()
        pltpu.make_async_copy(dpad_hbm.at[dstb.at[0, 0]], dt_v.at[b],
                              sems[b]).wait()
        pltpu.make_async_copy(hlo_hbm.at[srcb.at[0, 0]], rows_v.at[b],
                              sems[b]).wait()

    def scatter_chunk_start(b, ci):
        j = lax.rem(ci, SUP)
        s = lax.rem(lax.div(ci, SUP), 2)
        di = dstb.at[s, j]
        pltpu.make_async_copy(rows_v.at[b], acc_h.at[di],
                              ssems[b]).start(add=True)
        pltpu.make_async_copy(pb_v.at[b], acc_p.at[di],
                              ssems[b]).start(add=True)

    def scatter_chunk_drain(b):
        pltpu.make_async_copy(rows_v.at[b], acc_h.at[dstb.at[0, 0]],
                              ssems[b]).wait()
        pltpu.make_async_copy(pb_v.at[b], acc_p.at[dstb.at[0, 0]],
                              ssems[b]).wait()

    def compute_chunk(b):
        rv = rows_v.at[b]
        pv_ref = pb_v.at[b]
        sv_ref = st_v.at[b]
        dv_ref = dt_v.at[b]

        def pgrp(g, _):
            row_idx = g * L + iota
            for t in range(HHALF):
                colv = jnp.zeros((L,), jnp.int32) + (col0 + t)
                sv = plsc.load_gather(sv_ref, [row_idx, colv])
                dv = plsc.load_gather(dv_ref, [row_idx, colv])
                e = sv + dv
                e = jnp.maximum(e, 0.2 * e)
                p = jnp.exp(e)
                plsc.store_scatter(
                    pv_ref, [row_idx, jnp.full((L,), t, jnp.int32)], p)
            return 0

        lax.fori_loop(0, CH // L, pgrp, 0)

        def erow(e, _):
            ei = jnp.full((L,), e, jnp.int32)
            for t in range(HHALF):
                pv = plsc.load_gather(pv_ref, [ei, jnp.full((L,), t, jnp.int32)])
                for j in range(HID // L):
                    col = t * HID + j * L
                    rv[e, pl.ds(col, L)] = rv[e, pl.ds(col, L)] * pv
            return 0

        lax.fori_loop(0, CH, erow, 0)

    load_super(0)
    load_chunk(0, 0)

    def pair(k, _):
        for b in range(2):
            ci = 2 * k + b
            wait_chunk(b)

            @pl.when(ci >= 1)
            def _():
                scatter_chunk_drain(1 - b)

            @pl.when((lax.rem(ci + 1, SUP) == 0) & (ci + 1 < nchunks))
            def _():
                load_super(lax.div(ci + 1, SUP))

            @pl.when(ci + 1 < nchunks)
            def _():
                load_chunk(1 - b, ci + 1)

            compute_chunk(b)
            scatter_chunk_start(b, ci)
        return 0

    lax.fori_loop(0, nchunks // 2, pair, 0)
    scatter_chunk_drain((nchunks - 1) % 2)
    plsc.subcore_barrier()

    @pl.when(cid == 0)
    def _():
        pltpu.sync_copy(acc_h.at[pl.ds(rbase, ROWS_PER_TILE)],
                        unlo_hbm.at[pl.ds(rbase, ROWS_PER_TILE)])
        pltpu.sync_copy(acc_p.at[pl.ds(rbase, ROWS_PER_TILE)],
                        denlo_hbm.at[pl.ds(rbase, ROWS_PER_TILE)])

    @pl.when(cid == 1)
    def _():
        pltpu.sync_copy(acc_h.at[pl.ds(rbase, ROWS_PER_TILE)],
                        unhi_hbm.at[pl.ds(rbase, ROWS_PER_TILE)])
        pltpu.sync_copy(acc_p.at[pl.ds(rbase, ROWS_PER_TILE)],
                        denhi_hbm.at[pl.ds(rbase, ROWS_PER_TILE)])


def _tc2_body(unlo_ref, unhi_ref, denlo_ref, denhi_ref, b1_ref, w2_ref,
              a2_ref, h2_ref):
    segs = []
    for un, den in ((unlo_ref, denlo_ref), (unhi_ref, denhi_ref)):
        u = un[...]
        d = den[...]
        for t in range(HHALF):
            segs.append(u[:, t * HID:(t + 1) * HID]
                        / (d[:, t:t + 1] + 1e-16))
    agg = jnp.concatenate(segs, axis=1) + b1_ref[...]
    out1 = jnp.where(agg > 0, agg, jnp.exp(jnp.minimum(agg, 0.0)) - 1.0)
    h2 = jnp.dot(out1, w2_ref[...], preferred_element_type=jnp.float32)
    a2 = a2_ref[...]
    s2 = jnp.sum(h2 * a2[0:1, :], axis=1, keepdims=True)
    d2 = jnp.sum(h2 * a2[1:2, :], axis=1, keepdims=True)
    z14 = jnp.zeros((BN, C - 2), jnp.float32)
    h2_ref[...] = jnp.concatenate([h2, s2, d2, z14], axis=1)


def _tc2(unlo, unhi, denlo, denhi, b1, W2, a2):
    grid = (N // BN,)
    return pl.pallas_call(
        _tc2_body,
        grid=grid,
        in_specs=[
            pl.BlockSpec((BN, FHALF), lambda i: (i, 0)),
            pl.BlockSpec((BN, FHALF), lambda i: (i, 0)),
            pl.BlockSpec((BN, L), lambda i: (i, 0)),
            pl.BlockSpec((BN, L), lambda i: (i, 0)),
            pl.BlockSpec((1, HEADS * HID), lambda i: (0, 0)),
            pl.BlockSpec((HEADS * HID, C), lambda i: (0, 0)),
            pl.BlockSpec((2, C), lambda i: (0, 0)),
        ],
        out_specs=pl.BlockSpec((BN, 2 * C), lambda i: (i, 0)),
        out_shape=jax.ShapeDtypeStruct((N, 2 * C), jnp.float32),
    )(unlo, unhi, denlo, denhi, b1, W2, a2)


ACC2W = 2 * L


@functools.lru_cache(maxsize=None)
def _sc2_call():
    return pl.kernel(
        _sc2_body,
        mesh=_mesh(),
        out_type=[
            jax.ShapeDtypeStruct((N, ACC2W), jnp.float32),
            jax.ShapeDtypeStruct((N, ACC2W), jnp.float32),
        ],
        scratch_types=[
            pltpu.VMEM((2, CH, ACC2W), jnp.float32),
            pltpu.VMEM((2, CH, ACC2W), jnp.float32),
            pltpu.VMEM((2, CH, ACC2W), jnp.float32),
            pltpu.VMEM((2, SUP, CH), jnp.int32),
            pltpu.VMEM((2, SUP, CH), jnp.int32),
            pltpu.VMEM_SHARED((N, ACC2W), jnp.float32),
            pltpu.SemaphoreType.DMA,
            pltpu.SemaphoreType.DMA,
            pltpu.SemaphoreType.DMA,
            pltpu.SemaphoreType.DMA,
        ],
        compiler_params=pltpu.CompilerParams(use_tc_tiling_on_sc=False,
                                             needs_layout_passes=False),
    )


def _sc2_body(src_hbm, dst_hbm, h2_hbm,
              acc_a_hbm, acc_b_hbm,
              srows_v, drows_v, rows_v, srcb, dstb, acc, sem0, sem1,
              sem2, sem3):
    cid = lax.axis_index("c")
    sid = lax.axis_index("s")
    sems = (sem0, sem1)
    ssems = (sem2, sem3)

    _zero_f32(rows_v.at[0], CH, ACC2W)
    _zero_f32(rows_v.at[1], CH, ACC2W)
    rbase = sid * ROWS_PER_TILE
    for k in range(ROWS_PER_TILE // 25):
        pltpu.sync_copy(rows_v.at[0, pl.ds(0, 25)],
                        acc.at[pl.ds(rbase + k * 25, 25)])
    plsc.subcore_barrier()

    wid = cid * NS + sid
    crow0 = wid * (EDGES_PER_TILE_L2 // CH)
    nchunks = EDGES_PER_TILE_L2 // CH
    iota = lax.iota(jnp.int32, L)

    def load_super(m):
        row = crow0 + m * SUP
        s = lax.rem(m, 2)
        pltpu.sync_copy(src_hbm.at[pl.ds(row, SUP)], srcb.at[s])
        pltpu.sync_copy(dst_hbm.at[pl.ds(row, SUP)], dstb.at[s])

    def load_chunk(b, ci):
        j = lax.rem(ci, SUP)
        s = lax.rem(lax.div(ci, SUP), 2)
        pltpu.make_async_copy(h2_hbm.at[srcb.at[s, j]], srows_v.at[b],
                              sems[b]).start()
        pltpu.make_async_copy(h2_hbm.at[dstb.at[s, j]], drows_v.at[b],
                              sems[b]).start()

    def wait_chunk(b):
        pltpu.make_async_copy(h2_hbm.at[srcb.at[0, 0]], srows_v.at[b],
                              sems[b]).wait()
        pltpu.make_async_copy(h2_hbm.at[dstb.at[0, 0]], drows_v.at[b],
                              sems[b]).wait()

    def compute_chunk(b):
        sr = srows_v.at[b]
        dr = drows_v.at[b]
        rv = rows_v.at[b]

        def pgrp(g, _):
            row_idx = g * L + iota
            sv = plsc.load_gather(sr, [row_idx, jnp.full((L,), C, jnp.int32)])
            dv = plsc.load_gather(dr, [row_idx, jnp.full((L,), C + 1, jnp.int32)])
            e = sv + dv
            e = jnp.maximum(e, 0.2 * e)
            p = jnp.exp(e)
            plsc.store_scatter(rv, [row_idx, jnp.full((L,), C, jnp.int32)], p)
            return 0

        lax.fori_loop(0, CH // L, pgrp, 0)

        def erow(e, _):
            ei = jnp.full((L,), e, jnp.int32)
            pv = plsc.load_gather(rv, [ei, jnp.full((L,), C, jnp.int32)])
            rv[e, pl.ds(0, L)] = sr[e, pl.ds(0, L)] * pv
            return 0

        lax.fori_loop(0, CH, erow, 0)

    def scatter_chunk_start(b, ci):
        j = lax.rem(ci, SUP)
        s = lax.rem(lax.div(ci, SUP), 2)
        pltpu.make_async_copy(rows_v.at[b], acc.at[dstb.at[s, j]],
                              ssems[b]).start(add=True)

    def scatter_chunk_drain(b):
        pltpu.make_async_copy(rows_v.at[b], acc.at[dstb.at[0, 0]],
                              ssems[b]).wait()

    load_super(0)
    load_chunk(0, 0)

    def pair(k, _):
        for b in range(2):
            ci = 2 * k + b
            wait_chunk(b)

            @pl.when(ci >= 1)
            def _():
                scatter_chunk_drain(1 - b)

            @pl.when((lax.rem(ci + 1, SUP) == 0) & (ci + 1 < nchunks))
            def _():
                load_super(lax.div(ci + 1, SUP))

            @pl.when(ci + 1 < nchunks)
            def _():
                load_chunk(1 - b, ci + 1)

            compute_chunk(b)
            scatter_chunk_start(b, ci)
        return 0

    lax.fori_loop(0, nchunks // 2, pair, 0)
    if nchunks % 2:
        ci = nchunks - 1
        wait_chunk(0)
        scatter_chunk_drain(1)
        compute_chunk(0)
        scatter_chunk_start(0, ci)
    scatter_chunk_drain((nchunks - 1) % 2)
    plsc.subcore_barrier()

    @pl.when(cid == 0)
    def _():
        pltpu.sync_copy(acc.at[pl.ds(rbase, ROWS_PER_TILE)],
                        acc_a_hbm.at[pl.ds(rbase, ROWS_PER_TILE)])

    @pl.when(cid == 1)
    def _():
        pltpu.sync_copy(acc.at[pl.ds(rbase, ROWS_PER_TILE)],
                        acc_b_hbm.at[pl.ds(rbase, ROWS_PER_TILE)])


def _tc3_body(a_ref, b_ref, b2_ref, out_ref):
    ua = a_ref[...]
    ub = b_ref[...]
    un = ua[:, :C] + ub[:, :C]
    den = ua[:, C:C + 1] + ub[:, C:C + 1]
    o = un / (den + 1e-16) + b2_ref[...]
    m = jnp.max(o, axis=1, keepdims=True)
    lse = m + jnp.log(jnp.sum(jnp.exp(o - m), axis=1, keepdims=True))
    out_ref[...] = o - lse


def _tc3(acc_a, acc_b, b2):
    grid = (N // BN,)
    return pl.pallas_call(
        _tc3_body,
        grid=grid,
        in_specs=[
            pl.BlockSpec((BN, ACC2W), lambda i: (i, 0)),
            pl.BlockSpec((BN, ACC2W), lambda i: (i, 0)),
            pl.BlockSpec((1, C), lambda i: (0, 0)),
        ],
        out_specs=pl.BlockSpec((BN, C), lambda i: (i, 0)),
        out_shape=jax.ShapeDtypeStruct((N, C), jnp.float32),
    )(acc_a, acc_b, b2)


@jax.jit
def kernel(x, edge_index, W1, a_src1, a_dst1, b1, W2, a_src2, a_dst2, b2):
    src = edge_index[0].reshape(E // CH, CH)
    dst = edge_index[1].reshape(E // CH, CH)

    eye = jnp.eye(HEADS, dtype=jnp.float32)
    A1s = (a_src1[:, :, None] * eye[:, None, :]).reshape(HEADS * HID, HEADS)
    A1d = (a_dst1[:, :, None] * eye[:, None, :]).reshape(HEADS * HID, HEADS)

    hlo, hhi, spad, dpad = _tc1(x, W1, A1s, A1d)
    unlo, unhi, denlo, denhi = _sc1_call()(src, dst, hlo, hhi, spad, dpad)
    h2aug = _tc2(unlo, unhi, denlo, denhi, b1.reshape(1, HEADS * HID), W2,
                 jnp.concatenate([a_src2, a_dst2], axis=0))
    acc_a, acc_b = _sc2_call()(src, dst, h2aug)
    return _tc3(acc_a, acc_b, b2.reshape(1, C))
